# baseline (speedup 1.0000x reference)
import functools

import jax
import jax.numpy as jnp
from jax import lax
from jax.experimental import pallas as pl
from jax.experimental.pallas import tpu as pltpu
from jax.experimental.pallas import tpu_sc as plsc

B, N, F = 8, 2048, 256
T = 8192
NW = 32
TPB = NW // B
CH = 32
NCH = T // (TPB * CH)
FPT = T // TPB
VPC = CH // 16

_mesh = plsc.VectorSubcoreMesh(core_axis_name="c", subcore_axis_name="s")


@functools.partial(
    pl.kernel,
    out_type=[
        jax.ShapeDtypeStruct((B * T, F), jnp.float32),
        jax.ShapeDtypeStruct((B * T,), jnp.int32),
    ],
    mesh=_mesh,
    compiler_params=pltpu.CompilerParams(needs_layout_passes=False),
    scratch_types=[
        pltpu.VMEM((N,), jnp.int32),
        pltpu.VMEM((N,), jnp.int32),
        pltpu.VMEM((FPT,), jnp.int32),
        pltpu.VMEM((FPT,), jnp.int32),
        pltpu.VMEM((FPT,), jnp.int32),
        pltpu.VMEM((FPT,), jnp.float32),
        pltpu.VMEM((FPT,), jnp.float32),
        pltpu.VMEM((FPT,), jnp.float32),
        pltpu.VMEM((FPT,), jnp.int32),
        pltpu.VMEM((16,), jnp.int32),
        pltpu.VMEM((CH, F), jnp.float32),
        pltpu.VMEM((CH, F), jnp.float32),
        pltpu.VMEM((CH, F), jnp.float32),
        pltpu.VMEM((CH, F), jnp.float32),
        pltpu.VMEM((CH, F), jnp.float32),
        pltpu.VMEM((CH, F), jnp.float32),
        pltpu.VMEM((CH, F), jnp.float32),
        pltpu.VMEM((CH, F), jnp.float32),
        pltpu.VMEM((CH, F), jnp.float32),
        pltpu.VMEM((CH,), jnp.int32),
        pltpu.SemaphoreType.DMA,
        pltpu.SemaphoreType.DMA,
        pltpu.SemaphoreType.DMA,
        pltpu.SemaphoreType.DMA,
    ],
)
def _sc_interp(st, mi, en, dur, mf, frames_o, mask_o,
               dur_v, csum_v, idxl_v, idxg_v, roff_v, a_v, b_v, c_v,
               msk_v, mf_v,
               sA, mA, eA, oA, sB, mB, eB, oB, zbuf, zmask_v,
               isem, osem, zsem, msem):
    cid = lax.axis_index("c")
    sid = lax.axis_index("s")
    wid = cid * 16 + sid
    b = wid // TPB
    sub = wid % TPB

    pltpu.sync_copy(dur.at[b], dur_v)
    pltpu.sync_copy(mf, mf_v)

    def cs_step(i, carry):
        v = dur_v[pl.ds(i * 16, 16)]
        csum_v[pl.ds(i * 16, 16)] = jnp.cumsum(v) + carry
        return carry + jnp.sum(v)

    total = lax.fori_loop(0, N // 16, cs_step, jnp.int32(0))
    vt_vec = jnp.minimum(jnp.full((16,), total, jnp.int32), mf_v[...])
    vt_s = jnp.max(vt_vec)

    def z_step(i, _):
        for k in range(F // 16):
            zbuf[i, pl.ds(k * 16, 16)] = jnp.zeros((16,), jnp.float32)
        return 0

    lax.fori_loop(0, CH, z_step, 0)
    for k in range(CH // 16):
        zmask_v[pl.ds(k * 16, 16)] = jnp.zeros((16,), jnp.int32)

    num = jnp.maximum(vt_s - sub * CH, 0)
    nv = jnp.minimum(jnp.int32(NCH), (num + CH * TPB - 1) // (CH * TPB))

    def row0_of(j):
        return b * T + (sub + TPB * j) * CH

    def precompute_chunk(c):
        def pre_step(k, carry):
            p_min, p_max = carry
            i = c * VPC + k
            l = i * 16
            g = sub + TPB * c
            tvec = g * CH + (l - c * CH) + lax.iota(jnp.int32, 16)
            lo = jnp.zeros((16,), jnp.int32)
            hi = jnp.full((16,), N, jnp.int32)
            for _step in range(12):
                m = jnp.minimum((lo + hi) >> 1, N - 1)
                vals = plsc.load_gather(csum_v, [m])
                pred = vals <= tvec
                lo = jnp.where(pred, m + 1, lo)
                hi = jnp.where(pred, hi, m)
            idx_c = jnp.minimum(lo, N - 1)
            d = plsc.load_gather(dur_v, [idx_c])
            cs = plsc.load_gather(csum_v, [idx_c])
            p = tvec - (cs - d)
            half = d >> 1
            rem = d - half
            pf = p.astype(jnp.float32)
            halff = half.astype(jnp.float32)
            den1 = jnp.maximum(half - 1, 1).astype(jnp.float32)
            den2 = jnp.maximum(rem - 1, 1).astype(jnp.float32)
            t1 = jnp.where(half > 1, pf / den1, 0.0)
            t2 = jnp.where(rem > 1, (pf - halff) / den2, 0.0)
            s1 = p < half
            a3 = jnp.where(s1, 1.0 - t1, 0.0)
            b3 = jnp.where(s1, t1, 1.0 - t2)
            c3 = jnp.where(s1, 0.0, t2)
            a2 = jnp.where(p == 0, 1.0, 0.0)
            is1 = d == 1
            is2 = d == 2
            a = jnp.where(is1, 0.0, jnp.where(is2, a2, a3))
            bb = jnp.where(is1, 1.0, jnp.where(is2, 0.0, b3))
            cc = jnp.where(is1, 0.0, jnp.where(is2, 1.0 - a2, c3))
            valid = tvec < vt_vec
            vf = jnp.where(valid, 1.0, 0.0)
            sl = pl.ds(l, 16)
            a_v[sl] = a * vf
            b_v[sl] = bb * vf
            c_v[sl] = cc * vf
            msk_v[sl] = jnp.where(valid, 1, 0)
            idxl_v[sl] = idx_c
            return (jnp.minimum(p_min, jnp.min(idx_c)),
                    jnp.maximum(p_max, jnp.max(idx_c)))

        p_lo, p_hi = lax.fori_loop(
            0, VPC, pre_step, (jnp.int32(N), jnp.int32(0)))
        p_base = jnp.minimum(p_lo & ~jnp.int32(7), jnp.int32(N - CH))
        lin = (p_hi - p_base) < jnp.int32(CH)

        def off_step(k, _):
            l = (c * VPC + k) * 16
            sl = pl.ds(l, 16)
            idx_c = idxl_v[sl]
            w = (l - c * CH) + lax.iota(jnp.int32, 16)
            roff_v[sl] = jnp.where(lin, idx_c - p_base, w)
            idxg_v[sl] = b * N + idx_c
            return 0

        lax.fori_loop(0, VPC, off_step, 0)
        return lin, p_base

    def fire_mask(j):
        pltpu.async_copy(msk_v.at[pl.ds(j * CH, CH)],
                         mask_o.at[pl.ds(row0_of(j), CH)], msem)

    def pre_out(j, _):
        r0 = row0_of(j)

        @pl.when(j >= nv)
        def _zero_chunk():
            pltpu.async_copy(zbuf, frames_o.at[pl.ds(r0, CH)], zsem)
            pltpu.async_copy(zmask_v, mask_o.at[pl.ds(r0, CH)], msem)

        return 0

    lax.fori_loop(0, NCH, pre_out, 0)

    bufs = ((sA, mA, eA, oA), (sB, mB, eB, oB))

    def fire_in(j, quad, lin, p_base):
        @pl.when(lin)
        def _linear():
            base = pl.multiple_of(b * N + p_base, 8)
            pltpu.async_copy(st.at[pl.ds(base, CH)], quad[0], isem)
            pltpu.async_copy(mi.at[pl.ds(base, CH)], quad[1], isem)
            pltpu.async_copy(en.at[pl.ds(base, CH)], quad[2], isem)

        @pl.when(jnp.logical_not(lin))
        def _indirect():
            idx_sl = idxg_v.at[pl.ds(j * CH, CH)]
            pltpu.async_copy(st.at[idx_sl], quad[0], isem)
            pltpu.async_copy(mi.at[idx_sl], quad[1], isem)
            pltpu.async_copy(en.at[idx_sl], quad[2], isem)

    def drain_in(quad):
        pltpu.make_async_copy(st.at[pl.ds(0, CH)], quad[0], isem).wait()
        pltpu.make_async_copy(mi.at[pl.ds(0, CH)], quad[1], isem).wait()
        pltpu.make_async_copy(en.at[pl.ds(0, CH)], quad[2], isem).wait()

    def drain_out_one():
        pltpu.make_async_copy(frames_o.at[pl.ds(0, CH)], oA, osem).wait()

    def process(i, quad):
        sb, mb, eb, ob = quad
        loc = i * CH
        col0 = lax.iota(jnp.int32, 16)

        def f_step(fr, _c):
            iv = jnp.full((16,), loc + fr, jnp.int32)
            asp = plsc.load_gather(a_v, [iv])
            bsp = plsc.load_gather(b_v, [iv])
            csp = plsc.load_gather(c_v, [iv])
            rv = plsc.load_gather(roff_v, [iv])
            for k in range(F // 16):
                cols = col0 + (k * 16)
                sv = plsc.load_gather(sb, [rv, cols])
                mv = plsc.load_gather(mb, [rv, cols])
                ev = plsc.load_gather(eb, [rv, cols])
                ob[fr, pl.ds(k * 16, 16)] = sv * asp + mv * bsp + ev * csp
            return 0

        lax.fori_loop(0, CH, f_step, 0)
        pltpu.async_copy(ob, frames_o.at[pl.ds(row0_of(i), CH)], osem)

    @pl.when(nv > 0)
    def _prologue():
        lin0, pb0 = precompute_chunk(jnp.int32(0))
        fire_mask(jnp.int32(0))
        fire_in(jnp.int32(0), bufs[0], lin0, pb0)

    def ring_step(i, _):
        @pl.when(i >= 1)
        def _free_other():
            drain_out_one()

        for par in (0, 1):
            @pl.when(i % 2 == par)
            def _sub(par=par):
                @pl.when(i + 1 < nv)
                def _prefetch():
                    lin, pb = precompute_chunk(i + 1)
                    fire_mask(i + 1)
                    fire_in(i + 1, bufs[1 - par], lin, pb)

                drain_in(bufs[par])
                process(i, bufs[par])

        return 0

    lax.fori_loop(0, nv, ring_step, 0)

    @pl.when(nv > 0)
    def _last_out():
        drain_out_one()

    def drain_z(i, _):
        pltpu.make_async_copy(frames_o.at[pl.ds(0, CH)], zbuf, zsem).wait()
        return 0

    lax.fori_loop(0, jnp.int32(NCH) - nv, drain_z, 0)

    def drain_m(i, _):
        pltpu.make_async_copy(mask_o.at[pl.ds(0, CH)],
                              msk_v.at[pl.ds(0, CH)], msem).wait()
        return 0

    lax.fori_loop(0, NCH, drain_m, 0)


def kernel(start, mid, end, durations, max_frames):
    st = start.reshape(B * N, F)
    mi = mid.reshape(B * N, F)
    en = end.reshape(B * N, F)
    dur = durations.astype(jnp.int32)
    mf = jnp.full((16,), jnp.asarray(max_frames, jnp.int32))
    frames_flat, mask_i = _sc_interp(st, mi, en, dur, mf)
    frames = frames_flat.reshape(B, T, F)
    mask = mask_i.reshape(B, T) != 0
    return frames, mask


# speedup vs baseline: 1.4147x; 1.4147x over previous
import functools

import jax
import jax.numpy as jnp
from jax import lax
from jax.experimental import pallas as pl
from jax.experimental.pallas import tpu as pltpu
from jax.experimental.pallas import tpu_sc as plsc

B, N, F = 8, 2048, 256
T = 8192
NW = 32
TPB = NW // B
CH = 64
NCH = T // (TPB * CH)
FPT = T // TPB
VPC = CH // 16
ZR = 32

_mesh = plsc.VectorSubcoreMesh(core_axis_name="c", subcore_axis_name="s")


@functools.partial(
    pl.kernel,
    out_type=[
        jax.ShapeDtypeStruct((B * T, F), jnp.float32),
        jax.ShapeDtypeStruct((B * T,), jnp.int32),
    ],
    mesh=_mesh,
    compiler_params=pltpu.CompilerParams(needs_layout_passes=False),
    scratch_types=[
        pltpu.VMEM((N,), jnp.int32),
        pltpu.VMEM((N,), jnp.int32),
        pltpu.VMEM((2 * CH,), jnp.int32),
        pltpu.VMEM((2 * CH,), jnp.int32),
        pltpu.VMEM((2 * CH,), jnp.int32),
        pltpu.VMEM((2 * CH,), jnp.float32),
        pltpu.VMEM((2 * CH,), jnp.float32),
        pltpu.VMEM((2 * CH,), jnp.float32),
        pltpu.VMEM((2 * CH,), jnp.int32),
        pltpu.VMEM((16,), jnp.int32),
        pltpu.VMEM((3 * CH, F), jnp.float32),
        pltpu.VMEM((3 * CH, F), jnp.float32),
        pltpu.VMEM((CH, F), jnp.float32),
        pltpu.VMEM((ZR, F), jnp.float32),
        pltpu.VMEM((CH,), jnp.int32),
        pltpu.SemaphoreType.DMA,
        pltpu.SemaphoreType.DMA,
        pltpu.SemaphoreType.DMA,
        pltpu.SemaphoreType.DMA,
    ],
)
def _sc_interp(st, mi, en, dur, mf, frames_o, mask_o,
               dur_v, csum_v, idxl_v, idxg_v, srow_v, a_v, b_v, c_v,
               msk_v, mf_v, wA, wB, ob, zbuf, zmask_v,
               isem, osem, zsem, msem):
    cid = lax.axis_index("c")
    sid = lax.axis_index("s")
    wid = cid * 16 + sid
    b = wid // TPB
    sub = wid % TPB

    pltpu.sync_copy(dur.at[b], dur_v)
    pltpu.sync_copy(mf, mf_v)

    def cs_step(i, carry):
        v = dur_v[pl.ds(i * 16, 16)]
        csum_v[pl.ds(i * 16, 16)] = jnp.cumsum(v) + carry
        return carry + jnp.sum(v)

    total = lax.fori_loop(0, N // 16, cs_step, jnp.int32(0))
    vt_vec = jnp.minimum(jnp.full((16,), total, jnp.int32), mf_v[...])
    vt_s = jnp.max(vt_vec)

    def z_step(i, _):
        for k in range(F // 16):
            zbuf[i, pl.ds(k * 16, 16)] = jnp.zeros((16,), jnp.float32)
        return 0

    lax.fori_loop(0, ZR, z_step, 0)
    for k in range(CH // 16):
        zmask_v[pl.ds(k * 16, 16)] = jnp.zeros((16,), jnp.int32)

    num = jnp.maximum(vt_s - sub * CH, 0)
    nv = jnp.minimum(jnp.int32(NCH), (num + CH * TPB - 1) // (CH * TPB))

    def row0_of(j):
        return b * T + (sub + TPB * j) * CH

    def precompute_chunk(c):
        slot0 = (c % 2) * CH

        def pre_step(k, carry):
            p_min, p_max, oh_acc = carry
            l = slot0 + k * 16
            g = sub + TPB * c
            tvec = g * CH + k * 16 + lax.iota(jnp.int32, 16)
            lo = jnp.zeros((16,), jnp.int32)
            hi = jnp.full((16,), N, jnp.int32)
            for _step in range(12):
                m = jnp.minimum((lo + hi) >> 1, N - 1)
                vals = plsc.load_gather(csum_v, [m])
                pred = vals <= tvec
                lo = jnp.where(pred, m + 1, lo)
                hi = jnp.where(pred, hi, m)
            idx_c = jnp.minimum(lo, N - 1)
            d = plsc.load_gather(dur_v, [idx_c])
            cs = plsc.load_gather(csum_v, [idx_c])
            p = tvec - (cs - d)
            half = d >> 1
            rem = d - half
            pf = p.astype(jnp.float32)
            halff = half.astype(jnp.float32)
            den1 = jnp.maximum(half - 1, 1).astype(jnp.float32)
            den2 = jnp.maximum(rem - 1, 1).astype(jnp.float32)
            t1 = jnp.where(half > 1, pf / den1, 0.0)
            t2 = jnp.where(rem > 1, (pf - halff) / den2, 0.0)
            s1 = p < half
            a3 = jnp.where(s1, 1.0 - t1, 0.0)
            b3 = jnp.where(s1, t1, 1.0 - t2)
            c3 = jnp.where(s1, 0.0, t2)
            a2 = jnp.where(p == 0, 1.0, 0.0)
            is1 = d == 1
            is2 = d == 2
            a = jnp.where(is1, 0.0, jnp.where(is2, a2, a3))
            bb = jnp.where(is1, 1.0, jnp.where(is2, 0.0, b3))
            cc = jnp.where(is1, 0.0, jnp.where(is2, 1.0 - a2, c3))
            valid = tvec < vt_vec
            vf = jnp.where(valid, 1.0, 0.0)
            sl = pl.ds(l, 16)
            a_v[sl] = a * vf
            b_v[sl] = bb * vf
            c_v[sl] = cc * vf
            msk_v[sl] = jnp.where(valid, 1, 0)
            idxl_v[sl] = idx_c
            sel = (jnp.where(bb == 1.0, 1, 0) + jnp.where(cc == 1.0, 2, 0))
            srow_v[sl] = sel * CH
            oh = jnp.where(
                valid & ((a == 1.0) | (bb == 1.0) | (cc == 1.0)), 1, 0)
            return (jnp.minimum(p_min, jnp.min(idx_c)),
                    jnp.maximum(p_max, jnp.max(idx_c)),
                    jnp.minimum(oh_acc, jnp.min(oh)))

        p_lo, p_hi, oh_all = lax.fori_loop(
            0, VPC, pre_step, (jnp.int32(N), jnp.int32(0), jnp.int32(1)))
        p_base = jnp.minimum(p_lo & ~jnp.int32(7), jnp.int32(N - CH))
        lin = (p_hi - p_base) < jnp.int32(CH)
        oh_flag = oh_all == 1

        def off_step(k, _):
            sl = pl.ds(slot0 + k * 16, 16)
            idx_c = idxl_v[sl]
            w = k * 16 + lax.iota(jnp.int32, 16)
            roff = jnp.where(lin, idx_c - p_base, w)
            srow_v[sl] = srow_v[sl] + roff
            idxg_v[sl] = b * N + idx_c
            return 0

        lax.fori_loop(0, VPC, off_step, 0)
        return lin, p_base, oh_flag

    def fire_mask(j):
        pltpu.async_copy(msk_v.at[pl.ds((j % 2) * CH, CH)],
                         mask_o.at[pl.ds(row0_of(j), CH)], msem)

    def pre_out(j, _):
        r0 = row0_of(j)

        @pl.when(j >= nv)
        def _zero_chunk():
            pltpu.async_copy(zbuf, frames_o.at[pl.ds(r0, ZR)], zsem)
            pltpu.async_copy(zbuf, frames_o.at[pl.ds(r0 + ZR, ZR)], zsem)
            pltpu.async_copy(zmask_v, mask_o.at[pl.ds(r0, CH)], zsem)

        return 0

    lax.fori_loop(0, NCH, pre_out, 0)

    bufs = (wA, wB)

    def fire_in(j, wbuf, lin, p_base):
        @pl.when(lin)
        def _linear():
            base = pl.multiple_of(b * N + p_base, 8)
            pltpu.async_copy(st.at[pl.ds(base, CH)],
                             wbuf.at[pl.ds(0, CH)], isem)
            pltpu.async_copy(mi.at[pl.ds(base, CH)],
                             wbuf.at[pl.ds(CH, CH)], isem)
            pltpu.async_copy(en.at[pl.ds(base, CH)],
                             wbuf.at[pl.ds(2 * CH, CH)], isem)

        @pl.when(jnp.logical_not(lin))
        def _indirect():
            idx_sl = idxg_v.at[pl.ds((j % 2) * CH, CH)]
            pltpu.async_copy(st.at[idx_sl], wbuf.at[pl.ds(0, CH)], isem)
            pltpu.async_copy(mi.at[idx_sl], wbuf.at[pl.ds(CH, CH)], isem)
            pltpu.async_copy(en.at[idx_sl], wbuf.at[pl.ds(2 * CH, CH)], isem)

    def drain_in(wbuf):
        pltpu.make_async_copy(st.at[pl.ds(0, CH)],
                              wbuf.at[pl.ds(0, CH)], isem).wait()
        pltpu.make_async_copy(mi.at[pl.ds(0, CH)],
                              wbuf.at[pl.ds(CH, CH)], isem).wait()
        pltpu.make_async_copy(en.at[pl.ds(0, CH)],
                              wbuf.at[pl.ds(2 * CH, CH)], isem).wait()

    def drain_out_one():
        pltpu.make_async_copy(frames_o.at[pl.ds(0, CH)], ob, osem).wait()

    def process(i, wbuf, oh_flag):
        slot0 = (i % 2) * CH
        col0 = lax.iota(jnp.int32, 16)

        @pl.when(oh_flag)
        def _copy_rows():
            def f_step(fr, _c):
                iv = jnp.full((16,), slot0 + fr, jnp.int32)
                rv = plsc.load_gather(srow_v, [iv])
                for k in range(F // 16):
                    ob[fr, pl.ds(k * 16, 16)] = plsc.load_gather(
                        wbuf, [rv, col0 + (k * 16)])
                return 0

            lax.fori_loop(0, CH, f_step, 0)

        @pl.when(jnp.logical_not(oh_flag))
        def _blend_rows():
            def f_step(fr, _c):
                iv = jnp.full((16,), slot0 + fr, jnp.int32)
                asp = plsc.load_gather(a_v, [iv])
                bsp = plsc.load_gather(b_v, [iv])
                csp = plsc.load_gather(c_v, [iv])
                rv = plsc.load_gather(srow_v, [iv]) & jnp.int32(CH - 1)
                for k in range(F // 16):
                    cols = col0 + (k * 16)
                    sv = plsc.load_gather(wbuf, [rv, cols])
                    mv = plsc.load_gather(wbuf, [rv + CH, cols])
                    ev = plsc.load_gather(wbuf, [rv + 2 * CH, cols])
                    ob[fr, pl.ds(k * 16, 16)] = (
                        sv * asp + mv * bsp + ev * csp)
                return 0

            lax.fori_loop(0, CH, f_step, 0)

        pltpu.async_copy(ob, frames_o.at[pl.ds(row0_of(i), CH)], osem)

    @pl.when(nv > 0)
    def _prologue():
        lin0, pb0, _oh0 = precompute_chunk(jnp.int32(0))
        fire_mask(jnp.int32(0))
        fire_in(jnp.int32(0), bufs[0], lin0, pb0)

    def chunk_oh(i):
        def oh_step(k, acc):
            sl = pl.ds((i % 2) * CH + k * 16, 16)
            a = a_v[sl]
            bb = b_v[sl]
            cc = c_v[sl]
            vld = msk_v[sl]
            oh = jnp.where(
                (vld == 1) & ((a == 1.0) | (bb == 1.0) | (cc == 1.0)), 1, 0)
            return jnp.minimum(acc, jnp.min(oh))

        return lax.fori_loop(0, VPC, oh_step, jnp.int32(1)) == 1

    def ring_step(i, _):
        @pl.when(i >= 1)
        def _free_other():
            drain_out_one()

        for par in (0, 1):
            @pl.when(i % 2 == par)
            def _sub(par=par):
                @pl.when(i + 1 < nv)
                def _prefetch():
                    @pl.when(i >= 1)
                    def _free_mask_slot():
                        pltpu.make_async_copy(
                            mask_o.at[pl.ds(0, CH)],
                            msk_v.at[pl.ds(0, CH)], msem).wait()

                    lin, pb, _oh = precompute_chunk(i + 1)
                    fire_mask(i + 1)
                    fire_in(i + 1, bufs[1 - par], lin, pb)

                drain_in(bufs[par])
                oh_i = chunk_oh(i)
                process(i, bufs[par], oh_i)

        return 0

    lax.fori_loop(0, nv, ring_step, 0)

    @pl.when(nv > 0)
    def _last_out():
        drain_out_one()

    def drain_z(i, _):
        pltpu.make_async_copy(frames_o.at[pl.ds(0, ZR)], zbuf, zsem).wait()
        pltpu.make_async_copy(frames_o.at[pl.ds(0, ZR)], zbuf, zsem).wait()
        pltpu.make_async_copy(mask_o.at[pl.ds(0, CH)],
                              zmask_v, zsem).wait()
        return 0

    lax.fori_loop(0, jnp.int32(NCH) - nv, drain_z, 0)

    def drain_m(i, _):
        pltpu.make_async_copy(mask_o.at[pl.ds(0, CH)],
                              msk_v.at[pl.ds(0, CH)], msem).wait()
        return 0

    lax.fori_loop(0, jnp.minimum(nv, jnp.int32(2)), drain_m, 0)


def kernel(start, mid, end, durations, max_frames):
    st = start.reshape(B * N, F)
    mi = mid.reshape(B * N, F)
    en = end.reshape(B * N, F)
    dur = durations.astype(jnp.int32)
    mf = jnp.full((16,), jnp.asarray(max_frames, jnp.int32))
    frames_flat, mask_i = _sc_interp(st, mi, en, dur, mf)
    frames = frames_flat.reshape(B, T, F)
    mask = mask_i.reshape(B, T) != 0
    return frames, mask


# speedup vs baseline: 1.9111x; 1.3509x over previous
import functools

import jax
import jax.numpy as jnp
from jax import lax
from jax.experimental import pallas as pl
from jax.experimental.pallas import tpu as pltpu
from jax.experimental.pallas import tpu_sc as plsc

B, N, F = 8, 2048, 256
T = 8192
NW = 32
TPB = NW // B
CH = 64
NCH = T // (TPB * CH)
FPT = T // TPB

_mesh = plsc.VectorSubcoreMesh(core_axis_name="c", subcore_axis_name="s")


@functools.partial(
    pl.kernel,
    out_type=[
        jax.ShapeDtypeStruct((B * T, F), jnp.float32),
        jax.ShapeDtypeStruct((B * T,), jnp.int32),
    ],
    mesh=_mesh,
    compiler_params=pltpu.CompilerParams(needs_layout_passes=False),
    scratch_types=[
        pltpu.VMEM((N,), jnp.int32),
        pltpu.VMEM((N,), jnp.int32),
        pltpu.VMEM((FPT,), jnp.int32),
        pltpu.VMEM((FPT,), jnp.float32),
        pltpu.VMEM((FPT,), jnp.float32),
        pltpu.VMEM((FPT,), jnp.float32),
        pltpu.VMEM((FPT,), jnp.int32),
        pltpu.VMEM((16,), jnp.int32),
        pltpu.VMEM((CH, F), jnp.float32),
        pltpu.VMEM((CH, F), jnp.float32),
        pltpu.VMEM((CH, F), jnp.float32),
        pltpu.VMEM((CH, F), jnp.float32),
        pltpu.VMEM((CH, F), jnp.float32),
        pltpu.VMEM((CH, F), jnp.float32),
        pltpu.VMEM((CH, F), jnp.float32),
        pltpu.VMEM((CH,), jnp.int32),
        pltpu.SemaphoreType.DMA,
        pltpu.SemaphoreType.DMA,
        pltpu.SemaphoreType.DMA,
        pltpu.SemaphoreType.DMA,
    ],
)
def _sc_interp(st, mi, en, dur, mf, frames_o, mask_o,
               dur_v, csum_v, idxg_v, a_v, b_v, c_v, msk_v, mf_v,
               sbufA, mbufA, ebufA, sbufB, mbufB, ebufB, zbuf, zmask_v,
               isem, osem, zsem, msem):
    cid = lax.axis_index("c")
    sid = lax.axis_index("s")
    wid = cid * 16 + sid
    b = wid // TPB
    sub = wid % TPB

    pltpu.sync_copy(dur.at[b], dur_v)
    pltpu.sync_copy(mf, mf_v)

    def cs_step(i, carry):
        v = dur_v[pl.ds(i * 16, 16)]
        csum_v[pl.ds(i * 16, 16)] = jnp.cumsum(v) + carry
        return carry + jnp.sum(v)

    total = lax.fori_loop(0, N // 16, cs_step, jnp.int32(0))
    vt_vec = jnp.minimum(jnp.full((16,), total, jnp.int32), mf_v[...])
    vt_s = jnp.max(vt_vec)

    def z_step(i, _):
        for k in range(F // 16):
            zbuf[i, pl.ds(k * 16, 16)] = jnp.zeros((16,), jnp.float32)
        return 0

    lax.fori_loop(0, CH, z_step, 0)
    for k in range(CH // 16):
        zmask_v[pl.ds(k * 16, 16)] = jnp.zeros((16,), jnp.int32)

    def pre_step(i, _):
        l = i * 16
        j = l // CH
        w = l - j * CH
        g = sub + TPB * j
        tvec = g * CH + w + lax.iota(jnp.int32, 16)
        lo = jnp.zeros((16,), jnp.int32)
        hi = jnp.full((16,), N, jnp.int32)
        for _step in range(12):
            m = jnp.minimum((lo + hi) >> 1, N - 1)
            vals = plsc.load_gather(csum_v, [m])
            pred = vals <= tvec
            lo = jnp.where(pred, m + 1, lo)
            hi = jnp.where(pred, hi, m)
        idx_c = jnp.minimum(lo, N - 1)
        d = plsc.load_gather(dur_v, [idx_c])
        cs = plsc.load_gather(csum_v, [idx_c])
        p = tvec - (cs - d)
        half = d >> 1
        rem = d - half
        pf = p.astype(jnp.float32)
        halff = half.astype(jnp.float32)
        den1 = jnp.maximum(half - 1, 1).astype(jnp.float32)
        den2 = jnp.maximum(rem - 1, 1).astype(jnp.float32)
        t1 = jnp.where(half > 1, pf / den1, 0.0)
        t2 = jnp.where(rem > 1, (pf - halff) / den2, 0.0)
        s1 = p < half
        a3 = jnp.where(s1, 1.0 - t1, 0.0)
        b3 = jnp.where(s1, t1, 1.0 - t2)
        c3 = jnp.where(s1, 0.0, t2)
        a2 = jnp.where(p == 0, 1.0, 0.0)
        is1 = d == 1
        is2 = d == 2
        a = jnp.where(is1, 0.0, jnp.where(is2, a2, a3))
        bb = jnp.where(is1, 1.0, jnp.where(is2, 0.0, b3))
        c = jnp.where(is1, 0.0, jnp.where(is2, 1.0 - a2, c3))
        valid = tvec < vt_vec
        vf = jnp.where(valid, 1.0, 0.0)
        sl = pl.ds(l, 16)
        a_v[sl] = a * vf
        b_v[sl] = bb * vf
        c_v[sl] = c * vf
        msk_v[sl] = jnp.where(valid, 1, 0)
        idxg_v[sl] = b * N + idx_c
        return 0

    def precompute_chunk(c):
        lax.fori_loop(c * (CH // 16), (c + 1) * (CH // 16), pre_step, 0)

    num = jnp.maximum(vt_s - sub * CH, 0)
    nv = jnp.minimum(jnp.int32(NCH), (num + CH * TPB - 1) // (CH * TPB))

    def row0_of(j):
        return b * T + (sub + TPB * j) * CH

    def fire_mask(j):
        pltpu.async_copy(msk_v.at[pl.ds(j * CH, CH)],
                         mask_o.at[pl.ds(row0_of(j), CH)], msem)

    def pre_out(j, _):
        r0 = row0_of(j)

        @pl.when(j >= nv)
        def _zero_chunk():
            pltpu.async_copy(zbuf, frames_o.at[pl.ds(r0, CH)], zsem)
            pltpu.async_copy(zmask_v, mask_o.at[pl.ds(r0, CH)], msem)

        return 0

    lax.fori_loop(0, NCH, pre_out, 0)

    bufs = ((sbufA, mbufA, ebufA), (sbufB, mbufB, ebufB))

    def fire_in(j, trio):
        idx_sl = idxg_v.at[pl.ds(j * CH, CH)]
        pltpu.async_copy(st.at[idx_sl], trio[0], isem)
        pltpu.async_copy(mi.at[idx_sl], trio[1], isem)
        pltpu.async_copy(en.at[idx_sl], trio[2], isem)

    def drain_in(trio):
        pltpu.make_async_copy(st.at[pl.ds(0, CH)], trio[0], isem).wait()
        pltpu.make_async_copy(mi.at[pl.ds(0, CH)], trio[1], isem).wait()
        pltpu.make_async_copy(en.at[pl.ds(0, CH)], trio[2], isem).wait()

    def drain_out_one():
        pltpu.make_async_copy(frames_o.at[pl.ds(0, CH)], sbufA, osem).wait()

    def process(i, trio):
        sb, mb, eb = trio
        loc = i * CH

        def f_step(fr, _c):
            iv = jnp.full((16,), loc + fr, jnp.int32)
            asp = plsc.load_gather(a_v, [iv])
            bsp = plsc.load_gather(b_v, [iv])
            csp = plsc.load_gather(c_v, [iv])
            for k in range(F // 16):
                fsl = pl.ds(k * 16, 16)
                sb[fr, fsl] = (sb[fr, fsl] * asp + mb[fr, fsl] * bsp
                               + eb[fr, fsl] * csp)
            return 0

        lax.fori_loop(0, CH, f_step, 0)
        pltpu.async_copy(sb, frames_o.at[pl.ds(row0_of(i), CH)], osem)

    @pl.when(nv > 0)
    def _prologue():
        precompute_chunk(jnp.int32(0))
        fire_mask(jnp.int32(0))
        fire_in(0, bufs[0])

    def ring_step(i, _):
        @pl.when(i >= 1)
        def _free_other():
            drain_out_one()

        @pl.when(i + 1 < nv)
        def _pre_next():
            precompute_chunk(i + 1)
            fire_mask(i + 1)

        for par in (0, 1):
            @pl.when(i % 2 == par)
            def _sub(par=par):
                @pl.when(i + 1 < nv)
                def _prefetch():
                    fire_in(i + 1, bufs[1 - par])

                drain_in(bufs[par])
                process(i, bufs[par])

        return 0

    lax.fori_loop(0, nv, ring_step, 0)

    @pl.when(nv > 0)
    def _last_out():
        drain_out_one()

    def drain_z(i, _):
        pltpu.make_async_copy(frames_o.at[pl.ds(0, CH)], zbuf, zsem).wait()
        return 0

    lax.fori_loop(0, jnp.int32(NCH) - nv, drain_z, 0)

    def drain_m(i, _):
        pltpu.make_async_copy(mask_o.at[pl.ds(0, CH)],
                              msk_v.at[pl.ds(0, CH)], msem).wait()
        return 0

    lax.fori_loop(0, NCH, drain_m, 0)


def kernel(start, mid, end, durations, max_frames):
    st = start.reshape(B * N, F)
    mi = mid.reshape(B * N, F)
    en = end.reshape(B * N, F)
    dur = durations.astype(jnp.int32)
    mf = jnp.full((16,), jnp.asarray(max_frames, jnp.int32))
    frames_flat, mask_i = _sc_interp(st, mi, en, dur, mf)
    frames = frames_flat.reshape(B, T, F)
    mask = mask_i.reshape(B, T) != 0
    return frames, mask
